# baseline (speedup 1.0000x reference)
import functools

import jax
import jax.numpy as jnp
from jax import lax
from jax.experimental import pallas as pl
from jax.experimental.pallas import tpu as pltpu
from jax.experimental.pallas import tpu_sc as plsc

N = 10000
E = 160000
D_IN = 256
D_OUT = 256
H = 128
CH = 128
NC = 2
NS = 16
CPT = 80
NCHUNK_PAD = NS * CPT
E_PAD = NCHUNK_PAD * CH
NROW = 3
NIDX = 4
NPAD = 10016


_BR = 400


def _mm_body(x_ref, w_ref, o_ref):
    o_ref[...] = jnp.dot(x_ref[...], w_ref[...],
                         preferred_element_type=jnp.float32)


def _matmul(inputs, W):
    grid = (N // _BR, NC)
    return pl.pallas_call(
        _mm_body,
        grid=grid,
        in_specs=[
            pl.BlockSpec((_BR, D_IN), lambda i, j: (i, 0)),
            pl.BlockSpec((D_IN, H), lambda i, j: (0, j)),
        ],
        out_specs=pl.BlockSpec((_BR, H), lambda i, j: (j * (N // _BR) + i, 0)),
        out_shape=jax.ShapeDtypeStruct((NC * N, H), jnp.float32),
    )(inputs, W)


def _sc_body(xcat_hbm, src_hbm, dst_hbm, adj_hbm, out_hbm,
             src4, dst4, adj4, rows, acc,
             gsem0, gsem1, gsem2, ssem0, ssem1, ssem2,
             isem0, isem1, isem2, isem3):
    c = lax.axis_index("c")
    s = lax.axis_index("s")
    gsems = (gsem0, gsem1, gsem2)
    ssems = (ssem0, ssem1, ssem2)
    isems = (isem0, isem1, isem2, isem3)
    cN = c * N
    base_chunk = s * CPT

    zero16 = jnp.zeros((16,), jnp.float32)

    def zrow(i, carry):
        for j in range(H // 16):
            rows[0, i, pl.ds(j * 16, 16)] = zero16
        return carry

    lax.fori_loop(0, CH, zrow, 0)

    @pl.when(s < NS - 1)
    def _():
        for k in range(5):
            pltpu.sync_copy(rows.at[0], acc.at[pl.ds(s * 640 + k * CH, CH)])

    @pl.when(s == NS - 1)
    def _():
        for k in range(3):
            pltpu.sync_copy(rows.at[0], acc.at[pl.ds(9600 + k * CH, CH)])
        pltpu.sync_copy(rows.at[0, pl.ds(0, 32)], acc.at[pl.ds(9984, 32)])

    plsc.subcore_barrier()

    def idx_copies(k, p):
        eb = (base_chunk + k) * CH
        return (
            pltpu.make_async_copy(src_hbm.at[pl.ds(eb, CH)],
                                  src4.at[pl.ds(p * CH, CH)], isems[p]),
            pltpu.make_async_copy(dst_hbm.at[pl.ds(eb, CH)], dst4.at[p],
                                  isems[p]),
            pltpu.make_async_copy(adj_hbm.at[pl.ds(eb, CH)],
                                  adj4.at[pl.ds(p * CH, CH)], isems[p]),
        )

    def idx_start(k, p):
        for cp in idx_copies(k, p):
            cp.start()

    def idx_wait(k, p):
        for cp in idx_copies(k, p):
            cp.wait()

    def fold(p):
        for g in range(CH // 16):
            sl = pl.ds(p * CH + g * 16, 16)
            src4[sl] = src4[sl] + cN

    def gather_copy(b, p):
        return pltpu.make_async_copy(xcat_hbm.at[src4.at[pl.ds(p * CH, CH)]],
                                     rows.at[b], gsems[b])

    def scatter_copy(b, p):
        return pltpu.make_async_copy(rows.at[b], acc.at[dst4.at[p]],
                                     ssems[b])

    def scatter_start(b, p):
        pltpu.async_copy(rows.at[b], acc.at[dst4.at[p]], ssems[b], add=True)

    dims = lax.GatherDimensionNumbers(
        offset_dims=(), collapsed_slice_dims=(0,), start_index_map=(0,))

    def scale_chunk(b, p):
        def escale(e, cc):
            a16 = adj4[pl.ds(p * CH + (e & ~15), 16)]
            ae = lax.gather(
                a16, jnp.broadcast_to(e & 15, (16,))[:, None], dims, (1,),
                mode=lax.GatherScatterMode.PROMISE_IN_BOUNDS)
            for v in range(H // 16):
                sl = pl.ds(v * 16, 16)
                rows[b, e, sl] = rows[b, e, sl] * ae
            return cc

        lax.fori_loop(0, CH, escale, 0)

    idx_start(0, 0)
    idx_start(1, 1)
    idx_wait(0, 0)
    fold(0)
    gather_copy(0, 0).start()

    NSUP = 12

    def super_step(q, carry):
        for jj in range(NSUP):
            i = q * NSUP + jj
            b, bn = jj % NROW, (jj + 1) % NROW
            p, pn, pnn = jj % NIDX, (jj + 1) % NIDX, (jj + 2) % NIDX

            @pl.when(i + 1 < CPT)
            def _():
                idx_wait(i + 1, pn)
                fold(pn)

                if jj >= 2:
                    scatter_copy(bn, pnn).wait()
                else:
                    @pl.when(q > 0)
                    def _():
                        scatter_copy(bn, pnn).wait()

                gather_copy(bn, pn).start()

            @pl.when(i + 2 < CPT)
            def _():
                idx_start(i + 2, pnn)

            @pl.when(i < CPT)
            def _():
                gather_copy(b, p).wait()
                scale_chunk(b, p)
                scatter_start(b, p)

        return carry

    lax.fori_loop(0, -(-CPT // NSUP), super_step, 0)

    for i in range(CPT - NROW, CPT):
        scatter_copy(i % NROW, i % NIDX).wait()

    plsc.subcore_barrier()

    @pl.when(s < NS - 1)
    def _():
        pltpu.sync_copy(acc.at[pl.ds(s * 640, 640)],
                        out_hbm.at[c, pl.ds(s * 640, 640)])

    @pl.when(s == NS - 1)
    def _():
        pltpu.sync_copy(acc.at[pl.ds(9600, 416)],
                        out_hbm.at[c, pl.ds(9600, 416)])


def _sc_scatter(x_cat, edge_index, adj_vals):
    mesh = plsc.VectorSubcoreMesh(core_axis_name="c", subcore_axis_name="s")
    fn = functools.partial(
        pl.kernel,
        out_type=jax.ShapeDtypeStruct((NC, NPAD, H), jnp.float32),
        mesh=mesh,
        scratch_types=[
            pltpu.VMEM((NIDX * CH,), jnp.int32),
            pltpu.VMEM((NIDX, CH), jnp.int32),
            pltpu.VMEM((NIDX * CH,), jnp.float32),
            pltpu.VMEM((NROW, CH, H), jnp.float32),
            pltpu.VMEM_SHARED((NPAD, H), jnp.float32),
        ] + [pltpu.SemaphoreType.DMA] * (2 * NROW + NIDX),
    )(_sc_body)
    pad = E_PAD - E
    src_e = jnp.pad(edge_index[1], (0, pad))
    dst_e = jnp.pad(edge_index[0], (0, pad))
    adj_e = jnp.pad(adj_vals, (0, pad))
    return fn(x_cat, src_e, dst_e, adj_e)


_BR2 = 400


def _epi_body(y_ref, b_ref, o_ref):
    y = jnp.concatenate([y_ref[0], y_ref[1]], axis=1) + b_ref[...][None, :]
    y = jnp.maximum(y, 0.0)
    nrm = jnp.sqrt(jnp.sum(y * y, axis=1, keepdims=True))
    o_ref[...] = y / jnp.maximum(nrm, 1e-12)


def _epilogue(y_cat, b):
    return pl.pallas_call(
        _epi_body,
        grid=(N // _BR2,),
        in_specs=[
            pl.BlockSpec((NC, _BR2, H), lambda i: (0, i, 0)),
            pl.BlockSpec((D_OUT,), lambda i: (0,)),
        ],
        out_specs=pl.BlockSpec((_BR2, D_OUT), lambda i: (i, 0)),
        out_shape=jax.ShapeDtypeStruct((N, D_OUT), jnp.float32),
    )(y_cat, b)


def kernel(inputs, edge_index, adj_vals, W, b):
    x_cat = _matmul(inputs, W)
    y_cat = _sc_scatter(x_cat, edge_index, adj_vals)
    return _epilogue(y_cat, b)


# speedup vs baseline: 1.0285x; 1.0285x over previous
import functools

import jax
import jax.numpy as jnp
from jax import lax
from jax.experimental import pallas as pl
from jax.experimental.pallas import tpu as pltpu
from jax.experimental.pallas import tpu_sc as plsc

N = 10000
E = 160000
D_IN = 256
D_OUT = 256
H = 128
CH = 128
NC = 2
NS = 16
CPT = 80
NCHUNK_PAD = NS * CPT
E_PAD = NCHUNK_PAD * CH
NROW = 3
NIDX = 4
NPAD = 10016


_BR = 400


def _mm_body(x_ref, w_ref, o_ref):
    o_ref[...] = jnp.dot(x_ref[...], w_ref[...],
                         preferred_element_type=jnp.float32)


def _matmul(inputs, W):
    grid = (N // _BR, NC)
    return pl.pallas_call(
        _mm_body,
        grid=grid,
        in_specs=[
            pl.BlockSpec((_BR, D_IN), lambda i, j: (i, 0)),
            pl.BlockSpec((D_IN, H), lambda i, j: (0, j)),
        ],
        out_specs=pl.BlockSpec((_BR, H), lambda i, j: (j * (N // _BR) + i, 0)),
        out_shape=jax.ShapeDtypeStruct((NC * N, H), jnp.float32),
    )(inputs, W)


def _sc_body(xcat_hbm, src_hbm, dst_hbm, adj_hbm, out_hbm,
             src4, dst4, adj4, rows, acc,
             gsem0, gsem1, gsem2, ssem0, ssem1, ssem2,
             isem0, isem1, isem2, isem3):
    c = lax.axis_index("c")
    s = lax.axis_index("s")
    gsems = (gsem0, gsem1, gsem2)
    ssems = (ssem0, ssem1, ssem2)
    isems = (isem0, isem1, isem2, isem3)
    cN = c * N
    base_chunk = s * CPT

    zero16 = jnp.zeros((16,), jnp.float32)

    def zrow(i, carry):
        for j in range(H // 16):
            rows[0, i, pl.ds(j * 16, 16)] = zero16
        return carry

    lax.fori_loop(0, CH, zrow, 0)

    @pl.when(s < NS - 1)
    def _():
        for k in range(5):
            pltpu.sync_copy(rows.at[0], acc.at[pl.ds(s * 640 + k * CH, CH)])

    @pl.when(s == NS - 1)
    def _():
        for k in range(3):
            pltpu.sync_copy(rows.at[0], acc.at[pl.ds(9600 + k * CH, CH)])
        pltpu.sync_copy(rows.at[0, pl.ds(0, 32)], acc.at[pl.ds(9984, 32)])

    plsc.subcore_barrier()

    def idx_copies(k, p):
        eb = (base_chunk + k) * CH
        return (
            pltpu.make_async_copy(src_hbm.at[pl.ds(eb, CH)],
                                  src4.at[pl.ds(p * CH, CH)], isems[p]),
            pltpu.make_async_copy(dst_hbm.at[pl.ds(eb, CH)], dst4.at[p],
                                  isems[p]),
            pltpu.make_async_copy(adj_hbm.at[pl.ds(eb, CH)],
                                  adj4.at[pl.ds(p * CH, CH)], isems[p]),
        )

    def idx_start(k, p):
        for cp in idx_copies(k, p):
            cp.start()

    def idx_wait(k, p):
        for cp in idx_copies(k, p):
            cp.wait()

    def fold(p):
        for g in range(CH // 16):
            sl = pl.ds(p * CH + g * 16, 16)
            src4[sl] = src4[sl] + cN

    def gather_copy(b, p):
        return pltpu.make_async_copy(xcat_hbm.at[src4.at[pl.ds(p * CH, CH)]],
                                     rows.at[b], gsems[b])

    def scatter_copy(b, p):
        return pltpu.make_async_copy(rows.at[b], acc.at[dst4.at[p]],
                                     ssems[b])

    def scatter_start(b, p):
        pltpu.async_copy(rows.at[b], acc.at[dst4.at[p]], ssems[b], add=True)

    dims = lax.GatherDimensionNumbers(
        offset_dims=(), collapsed_slice_dims=(0,), start_index_map=(0,))

    def scale_chunk(b, p):
        def gscale(g, cc):
            a16 = adj4[pl.ds(p * CH + g * 16, 16)]
            for e in range(16):
                ae = lax.gather(
                    a16, jnp.full((16, 1), e, jnp.int32), dims, (1,),
                    mode=lax.GatherScatterMode.PROMISE_IN_BOUNDS)
                row = g * 16 + e
                for v in range(H // 16):
                    sl = pl.ds(v * 16, 16)
                    rows[b, row, sl] = rows[b, row, sl] * ae
            return cc

        lax.fori_loop(0, CH // 16, gscale, 0)

    idx_start(0, 0)
    idx_start(1, 1)
    idx_wait(0, 0)
    fold(0)
    gather_copy(0, 0).start()

    NSUP = 12

    def super_step(q, carry):
        for jj in range(NSUP):
            i = q * NSUP + jj
            b, bn = jj % NROW, (jj + 1) % NROW
            p, pn, pnn = jj % NIDX, (jj + 1) % NIDX, (jj + 2) % NIDX

            @pl.when(i + 1 < CPT)
            def _():
                idx_wait(i + 1, pn)
                fold(pn)

                if jj >= 2:
                    scatter_copy(bn, pnn).wait()
                else:
                    @pl.when(q > 0)
                    def _():
                        scatter_copy(bn, pnn).wait()

                gather_copy(bn, pn).start()

            @pl.when(i + 2 < CPT)
            def _():
                idx_start(i + 2, pnn)

            @pl.when(i < CPT)
            def _():
                gather_copy(b, p).wait()
                scale_chunk(b, p)
                scatter_start(b, p)

        return carry

    lax.fori_loop(0, -(-CPT // NSUP), super_step, 0)

    for i in range(CPT - NROW, CPT):
        scatter_copy(i % NROW, i % NIDX).wait()

    plsc.subcore_barrier()

    @pl.when(s < NS - 1)
    def _():
        pltpu.sync_copy(acc.at[pl.ds(s * 640, 640)],
                        out_hbm.at[c, pl.ds(s * 640, 640)])

    @pl.when(s == NS - 1)
    def _():
        pltpu.sync_copy(acc.at[pl.ds(9600, 416)],
                        out_hbm.at[c, pl.ds(9600, 416)])


def _sc_scatter(x_cat, edge_index, adj_vals):
    mesh = plsc.VectorSubcoreMesh(core_axis_name="c", subcore_axis_name="s")
    fn = functools.partial(
        pl.kernel,
        out_type=jax.ShapeDtypeStruct((NC, NPAD, H), jnp.float32),
        mesh=mesh,
        scratch_types=[
            pltpu.VMEM((NIDX * CH,), jnp.int32),
            pltpu.VMEM((NIDX, CH), jnp.int32),
            pltpu.VMEM((NIDX * CH,), jnp.float32),
            pltpu.VMEM((NROW, CH, H), jnp.float32),
            pltpu.VMEM_SHARED((NPAD, H), jnp.float32),
        ] + [pltpu.SemaphoreType.DMA] * (2 * NROW + NIDX),
    )(_sc_body)
    pad = E_PAD - E
    src_e = jnp.pad(edge_index[1], (0, pad))
    dst_e = jnp.pad(edge_index[0], (0, pad))
    adj_e = jnp.pad(adj_vals, (0, pad))
    return fn(x_cat, src_e, dst_e, adj_e)


_BR2 = 400


def _epi_body(y_ref, b_ref, o_ref):
    y = jnp.concatenate([y_ref[0], y_ref[1]], axis=1) + b_ref[...][None, :]
    y = jnp.maximum(y, 0.0)
    nrm = jnp.sqrt(jnp.sum(y * y, axis=1, keepdims=True))
    o_ref[...] = y / jnp.maximum(nrm, 1e-12)


def _epilogue(y_cat, b):
    return pl.pallas_call(
        _epi_body,
        grid=(N // _BR2,),
        in_specs=[
            pl.BlockSpec((NC, _BR2, H), lambda i: (0, i, 0)),
            pl.BlockSpec((D_OUT,), lambda i: (0,)),
        ],
        out_specs=pl.BlockSpec((_BR2, D_OUT), lambda i: (i, 0)),
        out_shape=jax.ShapeDtypeStruct((N, D_OUT), jnp.float32),
    )(y_cat, b)


def kernel(inputs, edge_index, adj_vals, W, b):
    x_cat = _matmul(inputs, W)
    y_cat = _sc_scatter(x_cat, edge_index, adj_vals)
    return _epilogue(y_cat, b)


# speedup vs baseline: 2.0217x; 1.9657x over previous
import functools

import jax
import jax.numpy as jnp
from jax import lax
from jax.experimental import pallas as pl
from jax.experimental.pallas import tpu as pltpu
from jax.experimental.pallas import tpu_sc as plsc

N = 10000
E = 160000
D_IN = 256
D_OUT = 256
H = 128
CH = 128
NC = 2
NS = 16
CPT = 80
CPT_LAST = E // CH - (NS - 1) * CPT
NROW = 3
NIDX = 4
NPAD = 10016


_BR = 400


def _mm_body(x_ref, w_ref, o_ref):
    o_ref[...] = jnp.dot(x_ref[...], w_ref[...],
                         preferred_element_type=jnp.float32)


def _matmul(inputs, W):
    grid = (N // _BR, NC)
    return pl.pallas_call(
        _mm_body,
        grid=grid,
        in_specs=[
            pl.BlockSpec((_BR, D_IN), lambda i, j: (i, 0)),
            pl.BlockSpec((D_IN, H), lambda i, j: (0, j)),
        ],
        out_specs=pl.BlockSpec((_BR, H), lambda i, j: (j * (N // _BR) + i, 0)),
        out_shape=jax.ShapeDtypeStruct((NC * N, H), jnp.float32),
    )(inputs, W)


def _sc_body(xcat_hbm, src_hbm, dst_hbm, adj_hbm, out_hbm,
             src4, dst4, adj4, rows, acc,
             gsem0, gsem1, gsem2, ssem0, ssem1, ssem2,
             isem0, isem1, isem2, isem3):
    c = lax.axis_index("c")
    s = lax.axis_index("s")
    gsems = (gsem0, gsem1, gsem2)
    ssems = (ssem0, ssem1, ssem2)
    isems = (isem0, isem1, isem2, isem3)
    cN = c * N
    base_chunk = s * CPT
    n = jnp.where(s == NS - 1, CPT_LAST, CPT)

    zero16 = jnp.zeros((16,), jnp.float32)

    def zrow(i, carry):
        for j in range(H // 16):
            rows[0, i, pl.ds(j * 16, 16)] = zero16
        return carry

    lax.fori_loop(0, CH, zrow, 0)

    @pl.when(s < NS - 1)
    def _():
        for k in range(5):
            pltpu.sync_copy(rows.at[0], acc.at[pl.ds(s * 640 + k * CH, CH)])

    @pl.when(s == NS - 1)
    def _():
        for k in range(3):
            pltpu.sync_copy(rows.at[0], acc.at[pl.ds(9600 + k * CH, CH)])
        pltpu.sync_copy(rows.at[0, pl.ds(0, 32)], acc.at[pl.ds(9984, 32)])

    plsc.subcore_barrier()

    def idx_copies(k, p):
        eb = (base_chunk + k) * CH
        return (
            pltpu.make_async_copy(src_hbm.at[pl.ds(eb, CH)],
                                  src4.at[pl.ds(p * CH, CH)], isems[p]),
            pltpu.make_async_copy(dst_hbm.at[pl.ds(eb, CH)], dst4.at[p],
                                  isems[p]),
            pltpu.make_async_copy(adj_hbm.at[pl.ds(eb, CH)],
                                  adj4.at[pl.ds(p * CH, CH)], isems[p]),
        )

    def idx_start(k, p):
        for cp in idx_copies(k, p):
            cp.start()

    def idx_wait(k, p):
        for cp in idx_copies(k, p):
            cp.wait()

    def fold(p):
        for g in range(CH // 16):
            sl = pl.ds(p * CH + g * 16, 16)
            src4[sl] = src4[sl] + cN

    def gather_copy(b, p):
        return pltpu.make_async_copy(xcat_hbm.at[src4.at[pl.ds(p * CH, CH)]],
                                     rows.at[b], gsems[b])

    def scatter_copy(b, p):
        return pltpu.make_async_copy(rows.at[b], acc.at[dst4.at[p]],
                                     ssems[b])

    def scatter_start(b, p):
        pltpu.async_copy(rows.at[b], acc.at[dst4.at[p]], ssems[b], add=True)

    dims = lax.GatherDimensionNumbers(
        offset_dims=(), collapsed_slice_dims=(0,), start_index_map=(0,))

    def scale_chunk(b, p):
        def gscale(g, cc):
            a16 = adj4[pl.ds(p * CH + g * 16, 16)]
            for e in range(16):
                ae = lax.gather(
                    a16, jnp.full((16, 1), e, jnp.int32), dims, (1,),
                    mode=lax.GatherScatterMode.PROMISE_IN_BOUNDS)
                row = g * 16 + e
                for v in range(H // 16):
                    sl = pl.ds(v * 16, 16)
                    rows[b, row, sl] = rows[b, row, sl] * ae
            return cc

        lax.fori_loop(0, CH // 16, gscale, 0)

    idx_start(0, 0)
    idx_start(1, 1)
    idx_wait(0, 0)
    fold(0)
    gather_copy(0, 0).start()

    NSUP = 12

    def super_step(q, carry):
        for jj in range(NSUP):
            i = q * NSUP + jj
            b, bn = jj % NROW, (jj + 1) % NROW
            p, pn, pnn = jj % NIDX, (jj + 1) % NIDX, (jj + 2) % NIDX

            @pl.when(i + 1 < n)
            def _():
                idx_wait(i + 1, pn)
                fold(pn)

                if jj >= 2:
                    scatter_copy(bn, pnn).wait()
                else:
                    @pl.when(q > 0)
                    def _():
                        scatter_copy(bn, pnn).wait()

                gather_copy(bn, pn).start()

            @pl.when(i + 2 < n)
            def _():
                idx_start(i + 2, pnn)

            @pl.when(i < n)
            def _():
                gather_copy(b, p).wait()
                scale_chunk(b, p)
                scatter_start(b, p)

        return carry

    lax.fori_loop(0, -(-CPT // NSUP), super_step, 0)

    for i in range(CPT - NROW, CPT):
        scatter_copy(i % NROW, i % NIDX).wait()

    plsc.subcore_barrier()

    @pl.when(s < NS - 1)
    def _():
        pltpu.sync_copy(acc.at[pl.ds(s * 640, 640)],
                        out_hbm.at[c, pl.ds(s * 640, 640)])

    @pl.when(s == NS - 1)
    def _():
        pltpu.sync_copy(acc.at[pl.ds(9600, 416)],
                        out_hbm.at[c, pl.ds(9600, 416)])


def _sc_scatter(x_cat, edge_index, adj_vals):
    mesh = plsc.VectorSubcoreMesh(core_axis_name="c", subcore_axis_name="s")
    fn = functools.partial(
        pl.kernel,
        out_type=jax.ShapeDtypeStruct((NC, NPAD, H), jnp.float32),
        mesh=mesh,
        scratch_types=[
            pltpu.VMEM((NIDX * CH,), jnp.int32),
            pltpu.VMEM((NIDX, CH), jnp.int32),
            pltpu.VMEM((NIDX * CH,), jnp.float32),
            pltpu.VMEM((NROW, CH, H), jnp.float32),
            pltpu.VMEM_SHARED((NPAD, H), jnp.float32),
        ] + [pltpu.SemaphoreType.DMA] * (2 * NROW + NIDX),
    )(_sc_body)
    return fn(x_cat, edge_index[1], edge_index[0], adj_vals)


_BR2 = 400


def _epi_body(y_ref, b_ref, o_ref):
    y = jnp.concatenate([y_ref[0], y_ref[1]], axis=1) + b_ref[...][None, :]
    y = jnp.maximum(y, 0.0)
    nrm = jnp.sqrt(jnp.sum(y * y, axis=1, keepdims=True))
    o_ref[...] = y / jnp.maximum(nrm, 1e-12)


def _epilogue(y_cat, b):
    return pl.pallas_call(
        _epi_body,
        grid=(N // _BR2,),
        in_specs=[
            pl.BlockSpec((NC, _BR2, H), lambda i: (0, i, 0)),
            pl.BlockSpec((D_OUT,), lambda i: (0,)),
        ],
        out_specs=pl.BlockSpec((_BR2, D_OUT), lambda i: (i, 0)),
        out_shape=jax.ShapeDtypeStruct((N, D_OUT), jnp.float32),
    )(y_cat, b)


def kernel(inputs, edge_index, adj_vals, W, b):
    x_cat = _matmul(inputs, W)
    y_cat = _sc_scatter(x_cat, edge_index, adj_vals)
    return _epilogue(y_cat, b)
